# initial kernel scaffold (unmeasured)
import jax
import jax.numpy as jnp
from jax import lax
from jax.experimental import pallas as pl
from jax.experimental.pallas import tpu as pltpu


def kernel(
    x,
):
    def body(*refs):
        pass

    out_shape = jax.ShapeDtypeStruct(..., jnp.float32)
    return pl.pallas_call(body, out_shape=out_shape)(...)


# baseline (device time: 230933 ns/iter reference)
import jax
import jax.numpy as jnp
from jax import lax
from jax.experimental import pallas as pl
from jax.experimental.pallas import tpu as pltpu

K = 1


def kernel(x):
    m, n = x.shape
    half = m // 2
    chunk = half // K
    x16 = x.astype(jnp.bfloat16)

    def body(x_ref, out_ref, dsend, drecv, fsend, frecv):
        my_x = lax.axis_index("x")
        my_y = lax.axis_index("y")
        my_z = lax.axis_index("z")
        xn = (1 - my_x, my_y, my_z)
        yn = (my_x, 1 - my_y, my_z)

        barrier = pltpu.get_barrier_semaphore()
        for nbr in (xn, yn):
            pl.semaphore_signal(
                barrier, inc=1, device_id=nbr,
                device_id_type=pl.DeviceIdType.MESH,
            )
        pl.semaphore_wait(barrier, 2)

        my_base = my_x * m
        rem_base = (1 - my_x) * m

        directs = []
        for k in range(K):
            off = my_y * half + k * chunk
            rd = pltpu.make_async_remote_copy(
                src_ref=x_ref.at[pl.ds(off, chunk), :],
                dst_ref=out_ref.at[pl.ds(my_base + off, chunk), :],
                send_sem=dsend.at[k],
                recv_sem=drecv.at[k],
                device_id=xn,
                device_id_type=pl.DeviceIdType.MESH,
            )
            rd.start()
            directs.append(rd)

        out_ref[pl.ds(my_base, m), :] = x_ref[...]

        fwds = []
        for k in range(K):
            directs[k].wait_recv()
            off = rem_base + my_y * half + k * chunk
            fw = pltpu.make_async_remote_copy(
                src_ref=out_ref.at[pl.ds(off, chunk), :],
                dst_ref=out_ref.at[pl.ds(off, chunk), :],
                send_sem=fsend.at[k],
                recv_sem=frecv.at[k],
                device_id=yn,
                device_id_type=pl.DeviceIdType.MESH,
            )
            fw.start()
            fwds.append(fw)

        for k in range(K):
            directs[k].wait_send()
            fwds[k].wait_send()
            fwds[k].wait_recv()

    return pl.pallas_call(
        body,
        out_shape=jax.ShapeDtypeStruct((2 * m, n), jnp.bfloat16),
        in_specs=[pl.BlockSpec(memory_space=pltpu.VMEM)],
        out_specs=pl.BlockSpec(memory_space=pltpu.VMEM),
        scratch_shapes=[
            pltpu.SemaphoreType.DMA((K,)),
            pltpu.SemaphoreType.DMA((K,)),
            pltpu.SemaphoreType.DMA((K,)),
            pltpu.SemaphoreType.DMA((K,)),
        ],
        compiler_params=pltpu.CompilerParams(collective_id=0),
    )(x16)


# device time: 147090 ns/iter; 1.5700x vs baseline; 1.5700x over previous
import jax
import jax.numpy as jnp
from jax import lax
from jax.experimental import pallas as pl
from jax.experimental.pallas import tpu as pltpu

K = 16


def kernel(x):
    m, n = x.shape
    half = m // 2
    chunk = half // K
    x16 = x.astype(jnp.bfloat16)

    def body(x_ref, out_ref, dsend, drecv, fsend, frecv):
        my_x = lax.axis_index("x")
        my_y = lax.axis_index("y")
        my_z = lax.axis_index("z")
        xn = (1 - my_x, my_y, my_z)
        yn = (my_x, 1 - my_y, my_z)

        barrier = pltpu.get_barrier_semaphore()
        for nbr in (xn, yn):
            pl.semaphore_signal(
                barrier, inc=1, device_id=nbr,
                device_id_type=pl.DeviceIdType.MESH,
            )
        pl.semaphore_wait(barrier, 2)

        my_base = my_x * m
        rem_base = (1 - my_x) * m

        directs = []
        for k in range(K):
            off = my_y * half + k * chunk
            rd = pltpu.make_async_remote_copy(
                src_ref=x_ref.at[pl.ds(off, chunk), :],
                dst_ref=out_ref.at[pl.ds(my_base + off, chunk), :],
                send_sem=dsend.at[k],
                recv_sem=drecv.at[k],
                device_id=xn,
                device_id_type=pl.DeviceIdType.MESH,
            )
            rd.start()
            directs.append(rd)

        out_ref[pl.ds(my_base, m), :] = x_ref[...]

        fwds = []
        for k in range(K):
            directs[k].wait_recv()
            off = rem_base + my_y * half + k * chunk
            fw = pltpu.make_async_remote_copy(
                src_ref=out_ref.at[pl.ds(off, chunk), :],
                dst_ref=out_ref.at[pl.ds(off, chunk), :],
                send_sem=fsend.at[k],
                recv_sem=frecv.at[k],
                device_id=yn,
                device_id_type=pl.DeviceIdType.MESH,
            )
            fw.start()
            fwds.append(fw)

        for k in range(K):
            directs[k].wait_send()
            fwds[k].wait_send()
            fwds[k].wait_recv()

    return pl.pallas_call(
        body,
        out_shape=jax.ShapeDtypeStruct((2 * m, n), jnp.bfloat16),
        in_specs=[pl.BlockSpec(memory_space=pltpu.VMEM)],
        out_specs=pl.BlockSpec(memory_space=pltpu.VMEM),
        scratch_shapes=[
            pltpu.SemaphoreType.DMA((K,)),
            pltpu.SemaphoreType.DMA((K,)),
            pltpu.SemaphoreType.DMA((K,)),
            pltpu.SemaphoreType.DMA((K,)),
        ],
        compiler_params=pltpu.CompilerParams(collective_id=0),
    )(x16)
